# Initial kernel scaffold; baseline (speedup 1.0000x reference)
import jax
import jax.numpy as jnp
from jax import lax
from jax.experimental import pallas as pl
from jax.experimental.pallas import tpu as pltpu
from jax.experimental.pallas import tpu_sc as plsc

_B = 16384
_F = 26
_D = 32

_NC = 2
_NS = 16
_NW = _NC * _NS
_ROWS_PER_W = _B // _NW
_ROWS_PER_CHUNK = 4
_IDX_PER_CHUNK = _ROWS_PER_CHUNK * _F
_NCHUNK = _ROWS_PER_W // _ROWS_PER_CHUNK
_SCALE = 1.0 / _F


def _accumulate(buf, out_v, g):
    for r in range(_ROWS_PER_CHUNK):
        base = r * _F
        acc0 = buf[base, pl.ds(0, 16)]
        acc1 = buf[base, pl.ds(16, 16)]
        for j in range(1, _F):
            acc0 = acc0 + buf[base + j, pl.ds(0, 16)]
            acc1 = acc1 + buf[base + j, pl.ds(16, 16)]
        row = g * _ROWS_PER_CHUNK + r
        out_v[row, pl.ds(0, 16)] = acc0 * _SCALE
        out_v[row, pl.ds(16, 16)] = acc1 * _SCALE


def _sc_kernel(idx_hbm, table_hbm, out_hbm, idx_v, buf0, buf1, out_v,
               sem0, sem1):
    wid = lax.axis_index("s") * _NC + lax.axis_index("c")
    pltpu.sync_copy(idx_hbm.at[pl.ds(wid * _NCHUNK, _NCHUNK)], idx_v)

    pltpu.async_copy(table_hbm.at[idx_v.at[0]], buf0, sem0)

    def loop_body(i, carry):
        g = 2 * i
        pltpu.async_copy(table_hbm.at[idx_v.at[g + 1]], buf1, sem1)
        pltpu.make_async_copy(table_hbm.at[idx_v.at[0]], buf0, sem0).wait()
        _accumulate(buf0, out_v, g)

        @pl.when(g + 2 < _NCHUNK)
        def _():
            pltpu.async_copy(table_hbm.at[idx_v.at[g + 2]], buf0, sem0)

        pltpu.make_async_copy(table_hbm.at[idx_v.at[0]], buf1, sem1).wait()
        _accumulate(buf1, out_v, g + 1)
        return carry

    lax.fori_loop(0, _NCHUNK // 2, loop_body, 0)

    pltpu.sync_copy(out_v, out_hbm.at[pl.ds(wid * _ROWS_PER_W, _ROWS_PER_W)])


@jax.jit
def kernel(indices, embedding_w):
    idx = indices.astype(jnp.int32).reshape(_NW * _NCHUNK, _IDX_PER_CHUNK)
    mesh = plsc.VectorSubcoreMesh(core_axis_name="c", subcore_axis_name="s")
    run = pl.kernel(
        _sc_kernel,
        out_type=jax.ShapeDtypeStruct((_B, _D), jnp.float32),
        mesh=mesh,
        scratch_types=[
            pltpu.VMEM((_NCHUNK, _IDX_PER_CHUNK), jnp.int32),
            pltpu.VMEM((_IDX_PER_CHUNK, _D), jnp.float32),
            pltpu.VMEM((_IDX_PER_CHUNK, _D), jnp.float32),
            pltpu.VMEM((_ROWS_PER_W, _D), jnp.float32),
            pltpu.SemaphoreType.DMA,
            pltpu.SemaphoreType.DMA,
        ],
    )
    return run(idx, embedding_w)


# speedup vs baseline: 1.8793x; 1.8793x over previous
import jax
import jax.numpy as jnp
from jax import lax
from jax.experimental import pallas as pl
from jax.experimental.pallas import tpu as pltpu
from jax.experimental.pallas import tpu_sc as plsc

_B = 16384
_F = 26
_D = 32

_NC = 2
_NS = 16
_NW = _NC * _NS
_ROWS_PER_W = _B // _NW
_ROWS_PER_CHUNK = 4
_IDX_PER_CHUNK = _ROWS_PER_CHUNK * _F
_NCHUNK = _ROWS_PER_W // _ROWS_PER_CHUNK
_SCALE = 1.0 / _F


def _accumulate(buf, out_v, g):
    for r in range(_ROWS_PER_CHUNK):
        base = r * _F
        acc0 = buf[base, pl.ds(0, 16)]
        acc1 = buf[base, pl.ds(16, 16)]
        for j in range(1, _F):
            acc0 = acc0 + buf[base + j, pl.ds(0, 16)]
            acc1 = acc1 + buf[base + j, pl.ds(16, 16)]
        row = g * _ROWS_PER_CHUNK + r
        out_v[row, pl.ds(0, 16)] = acc0 * _SCALE
        out_v[row, pl.ds(16, 16)] = acc1 * _SCALE


def _sc_kernel(idx_hbm, table_hbm, out_hbm, idx_v, buf0, buf1, out_v,
               sem0, sem1):
    wid = lax.axis_index("s") * _NC + lax.axis_index("c")
    pltpu.sync_copy(idx_hbm.at[pl.ds(wid * _NCHUNK, _NCHUNK)], idx_v)

    pltpu.async_copy(table_hbm.at[idx_v.at[0]], buf0, sem0)

    def loop_body(i, carry):
        g = 2 * i
        pltpu.async_copy(table_hbm.at[idx_v.at[g + 1]], buf1, sem1)
        pltpu.make_async_copy(table_hbm.at[idx_v.at[0]], buf0, sem0).wait()
        _accumulate(buf0, out_v, g)

        @pl.when(g + 2 < _NCHUNK)
        def _():
            pltpu.async_copy(table_hbm.at[idx_v.at[g + 2]], buf0, sem0)

        pltpu.make_async_copy(table_hbm.at[idx_v.at[0]], buf1, sem1).wait()
        _accumulate(buf1, out_v, g + 1)
        return carry

    lax.fori_loop(0, _NCHUNK // 2, loop_body, 0)

    pltpu.sync_copy(out_v, out_hbm.at[pl.ds(wid * _ROWS_PER_W, _ROWS_PER_W)])


@jax.jit
def kernel(indices, embedding_w):
    idx = indices.astype(jnp.int32).reshape(_NW * _NCHUNK, _IDX_PER_CHUNK)
    mesh = plsc.VectorSubcoreMesh(core_axis_name="c", subcore_axis_name="s")
    run = pl.kernel(
        _sc_kernel,
        out_type=jax.ShapeDtypeStruct((_B, _D), jnp.float32),
        mesh=mesh,
        scratch_types=[
            pltpu.VMEM((_NCHUNK, _IDX_PER_CHUNK), jnp.int32),
            pltpu.VMEM((_IDX_PER_CHUNK, _D), jnp.float32),
            pltpu.VMEM((_IDX_PER_CHUNK, _D), jnp.float32),
            pltpu.VMEM((_ROWS_PER_W, _D), jnp.float32),
            pltpu.SemaphoreType.DMA,
            pltpu.SemaphoreType.DMA,
        ],
        compiler_params=pltpu.CompilerParams(use_tc_tiling_on_sc=False),
    )
    return run(idx, embedding_w)
